# baseline (speedup 1.0000x reference)
import functools

import jax
import jax.numpy as jnp
from jax import lax
from jax.experimental import pallas as pl
from jax.experimental.pallas import tpu as pltpu
from jax.experimental.pallas import tpu_sc as plsc

B = 4096
N = 50
D = 64
V = 1000000
NTC = (V + 127) // 128
VT = NTC * 128
NT = 3
IDX_MINOR = 100


def _sc_transpose(nc, ns, nl):
    nw = nc * ns
    tpw = NT * ((NTC + NT * nw - 1) // (NT * nw))
    nblk = tpw // NT
    mesh = plsc.VectorSubcoreMesh(core_axis_name="c", subcore_axis_name="s")

    @functools.partial(
        pl.kernel,
        out_type=jax.ShapeDtypeStruct((VT * D,), jnp.float32),
        mesh=mesh,
        compiler_params=pltpu.CompilerParams(
            needs_layout_passes=False, use_tc_tiling_on_sc=True),
        scratch_types=[
            pltpu.VMEM((NT, 8, 8, 128), jnp.float32),
            pltpu.VMEM((NT, 8, 8, 128), jnp.float32),
            pltpu.VMEM((NT * 128 * D,), jnp.float32),
            pltpu.VMEM((NT * 128 * D,), jnp.float32),
            pltpu.SemaphoreType.DMA,
            pltpu.SemaphoreType.DMA,
            pltpu.SemaphoreType.DMA,
            pltpu.SemaphoreType.DMA,
        ],
    )
    def sc_transpose(tab_hbm, out_hbm, in_a, in_b, out_a, out_b,
                     sem_ia, sem_ib, sem_oa, sem_ob):
        wid = lax.axis_index("s") * nc + lax.axis_index("c")
        lane = lax.iota(jnp.int32, nl)
        l_hi = lane >> 3
        l_lo = lane & 7
        t_begin = wid * tpw
        t_end = jnp.minimum(t_begin + tpw, NTC)
        ins = (in_a, in_b)
        outs = (out_a, out_b)
        sem_i = (sem_ia, sem_ib)
        sem_o = (sem_oa, sem_ob)

        def c0_of(t):
            return pl.multiple_of(t * 128, 128)

        def in_copies(b, p):
            t0 = t_begin + b * NT
            res = []
            for jt in range(NT):
                c0 = c0_of(t0 + jt)
                for dt in range(8):
                    res.append((
                        tab_hbm.at[pl.ds(dt * 8, 8), pl.ds(c0, 128)],
                        ins[p].at[jt, dt],
                    ))
            return res, t0

        def fire_in(b, p):
            pairs, t0 = in_copies(b, p)
            for jt in range(NT):
                @pl.when(t0 + jt < t_end)
                def _():
                    for src, dst in pairs[jt * 8:(jt + 1) * 8]:
                        pltpu.async_copy(src, dst, sem_i[p])

        def drain_in(b, p):
            pairs, t0 = in_copies(b, p)
            for jt in range(NT):
                @pl.when(t0 + jt < t_end)
                def _():
                    for src, dst in pairs[jt * 8:(jt + 1) * 8]:
                        pltpu.make_async_copy(src, dst, sem_i[p]).wait()

        def out_copies(b, p):
            t0 = t_begin + b * NT
            res = []
            for jt in range(NT):
                c0 = c0_of(t0 + jt)
                res.append((
                    outs[p].at[pl.ds(jt * 128 * D, 128 * D)],
                    out_hbm.at[pl.ds(c0 * D, 128 * D)],
                ))
            return res, t0

        def fire_out(b, p):
            pairs, t0 = out_copies(b, p)
            for jt in range(NT):
                @pl.when(t0 + jt < t_end)
                def _():
                    pltpu.async_copy(pairs[jt][0], pairs[jt][1], sem_o[p])

        def drain_out(b, p):
            pairs, t0 = out_copies(b, p)
            for jt in range(NT):
                @pl.when((b >= 0) & (t0 + jt < t_end))
                def _():
                    pltpu.make_async_copy(pairs[jt][0], pairs[jt][1],
                                          sem_o[p]).wait()

        def compute(p):
            dtv = [l_hi + 2 * k for k in range(D // nl)]

            for jt in range(NT):
                jtv = jnp.full((nl,), jt, jnp.int32)

                def col_body(jl, _):
                    jlv = jnp.zeros((nl,), jnp.int32) + jl
                    base = jt * 128 * D + jl * D
                    for k in range(D // nl):
                        vals = plsc.load_gather(
                            ins[p], [jtv, dtv[k], l_lo, jlv])
                        outs[p][pl.ds(base + k * nl, nl)] = vals
                    return 0

                lax.fori_loop(0, 128, col_body, 0)

        fire_in(0, 0)

        def pair_body(bb, _):
            b0 = 2 * bb
            fire_in(b0 + 1, 1)
            drain_in(b0, 0)
            drain_out(b0 - 2, 0)
            compute(0)
            fire_out(b0, 0)

            @pl.when(bb < nblk // 2 - 1)
            def _():
                fire_in(b0 + 2, 0)

            drain_in(b0 + 1, 1)
            drain_out(b0 - 1, 1)
            compute(1)
            fire_out(b0 + 1, 1)
            return 0

        lax.fori_loop(0, nblk // 2, pair_body, 0)
        drain_out(nblk - 2, 0)
        drain_out(nblk - 1, 1)

    return sc_transpose


def _entity_kernel(nc, ns, nl):
    nw = nc * ns
    rows_per_w = B // nw
    cb = nl
    n_chunks = rows_per_w // cb
    g_rows = cb * N
    n_sub = g_rows // IDX_MINOR
    nq = (N + nl - 1) // nl
    tail = N - (nq - 1) * nl
    mesh = plsc.VectorSubcoreMesh(core_axis_name="c", subcore_axis_name="s")

    @functools.partial(
        pl.kernel,
        out_type=(
            jax.ShapeDtypeStruct((B * N,), jnp.float32),
            jax.ShapeDtypeStruct((B * N,), jnp.float32),
        ),
        mesh=mesh,
        compiler_params=pltpu.CompilerParams(
            needs_layout_passes=False, use_tc_tiling_on_sc=False),
        scratch_types=[
            pltpu.VMEM((2, n_sub, IDX_MINOR), jnp.int32),
            pltpu.VMEM((2, g_rows, D), jnp.float32),
            pltpu.VMEM((2, cb, D), jnp.float32),
            pltpu.VMEM((g_rows,), jnp.float32),
            pltpu.VMEM((g_rows,), jnp.float32),
            pltpu.SemaphoreType.DMA,
            pltpu.SemaphoreType.DMA,
        ],
    )
    def entity_kernel(ids_hbm, ctx_hbm, table_hbm, scores_hbm, post_hbm,
                      idx_v, rows_v, ctx_v, fs_v, fp_v, sem_a, sem_b):
        wid = lax.axis_index("s") * nc + lax.axis_index("c")
        lane = lax.iota(jnp.int32, nl)
        neg_inf = jnp.float32(-jnp.inf)
        sems = (sem_a, sem_b)

        def fire(c, p):
            pltpu.sync_copy(ids_hbm.at[pl.ds(c * n_sub, n_sub)], idx_v.at[p])
            handles = [
                pltpu.async_copy(
                    table_hbm.at[idx_v.at[p, j]],
                    rows_v.at[p, pl.ds(j * IDX_MINOR, IDX_MINOR)],
                    sems[p],
                )
                for j in range(n_sub)
            ]
            pltpu.sync_copy(ctx_hbm.at[pl.ds(c * cb, cb)], ctx_v.at[p])
            return handles

        def drain(p):
            for j in range(n_sub):
                pltpu.make_async_copy(
                    table_hbm.at[idx_v.at[p, j]],
                    rows_v.at[p, pl.ds(j * IDX_MINOR, IDX_MINOR)],
                    sems[p],
                ).wait()

        def compute_chunk(c, p):
            fbase = c * g_rows

            def row_body(b, _):
                rbase = b * N
                obase = b * N
                cvec = [ctx_v[p, b, pl.ds(k * nl, nl)]
                        for k in range(D // nl)]

                svs = []
                for q in range(nq):
                    acc = jnp.zeros((nl,), jnp.float32)
                    nlim = tail if q == nq - 1 else nl
                    for j in range(nlim):
                        r = rbase + q * nl + j
                        prod = rows_v[p, r, pl.ds(0, nl)] * cvec[0]
                        for k in range(1, D // nl):
                            prod = (prod
                                    + rows_v[p, r, pl.ds(k * nl, nl)] * cvec[k])
                        acc = jnp.where(lane == j, jnp.sum(prod), acc)
                    svs.append(acc)

                mvec = jnp.where(lane < tail, svs[nq - 1], neg_inf)
                for q in range(nq - 1):
                    mvec = jnp.maximum(mvec, svs[q])
                m = jnp.max(mvec)
                evs = [jnp.exp(sv - m) for sv in svs]
                evs[nq - 1] = jnp.where(lane < tail, evs[nq - 1], 0.0)
                ssum = evs[0]
                for q in range(1, nq):
                    ssum = ssum + evs[q]
                svec = jnp.zeros((nl,), jnp.float32) + jnp.sum(ssum)
                rinv = jnp.ones((nl,), jnp.float32) / svec

                for q in range(nq - 1):
                    fs_v[pl.ds(obase + q * nl, nl)] = svs[q]
                    fp_v[pl.ds(obase + q * nl, nl)] = evs[q] * rinv
                tmask = lane < tail
                tidx = obase + (nq - 1) * nl + lane
                plsc.store_scatter(fs_v, [tidx], svs[nq - 1], mask=tmask)
                plsc.store_scatter(fp_v, [tidx], evs[nq - 1] * rinv,
                                   mask=tmask)
                return 0

            lax.fori_loop(0, cb, row_body, 0)

            pltpu.sync_copy(fs_v, scores_hbm.at[pl.ds(fbase, g_rows)])
            pltpu.sync_copy(fp_v, post_hbm.at[pl.ds(fbase, g_rows)])

        c0_first = wid * n_chunks
        fire(c0_first, 0)

        def pair_body(gg, _):
            c = c0_first + 2 * gg
            handles_b = fire(c + 1, 1)
            drain(0)
            compute_chunk(c, 0)

            @pl.when(gg < n_chunks // 2 - 1)
            def _():
                fire(c + 2, 0)

            for h in handles_b:
                h.wait()
            compute_chunk(c + 1, 1)
            return 0

        lax.fori_loop(0, n_chunks // 2, pair_body, 0)

    return entity_kernel


def kernel(context_encoded, entity_ids, knwn_entity_embeddings):
    info = plsc.get_sparse_core_info()
    nc, ns, nl = info.num_cores, info.num_subcores, info.num_lanes
    tr = _sc_transpose(nc, ns, nl)
    table_lin = tr(knwn_entity_embeddings.T).reshape(VT, D)
    ids2d = entity_ids.astype(jnp.int32).reshape(B * N // IDX_MINOR,
                                                 IDX_MINOR)
    k = _entity_kernel(nc, ns, nl)
    scores_flat, post_flat = k(ids2d, context_encoded, table_lin)
    return scores_flat.reshape(B, N), post_flat.reshape(B, N)


# speedup vs baseline: 1.2139x; 1.2139x over previous
import functools

import jax
import jax.numpy as jnp
from jax import lax
from jax.experimental import pallas as pl
from jax.experimental.pallas import tpu as pltpu
from jax.experimental.pallas import tpu_sc as plsc

B = 4096
N = 50
D = 64
V = 1000000
NTC = (V + 127) // 128
VT = NTC * 128
NT = 3
IDX_MINOR = 100


def _sc_transpose(nc, ns, nl):
    nw = nc * ns
    tpw = NT * ((NTC + NT * nw - 1) // (NT * nw))
    nblk = tpw // NT
    mesh = plsc.VectorSubcoreMesh(core_axis_name="c", subcore_axis_name="s")

    @functools.partial(
        pl.kernel,
        out_type=jax.ShapeDtypeStruct((VT * D,), jnp.float32),
        mesh=mesh,
        compiler_params=pltpu.CompilerParams(
            needs_layout_passes=False, use_tc_tiling_on_sc=True),
        scratch_types=[
            pltpu.VMEM((NT, 8, 8, 128), jnp.float32),
            pltpu.VMEM((NT, 8, 8, 128), jnp.float32),
            pltpu.VMEM((NT * 128 * D,), jnp.float32),
            pltpu.VMEM((NT * 128 * D,), jnp.float32),
            pltpu.SemaphoreType.DMA,
            pltpu.SemaphoreType.DMA,
            pltpu.SemaphoreType.DMA,
            pltpu.SemaphoreType.DMA,
        ],
    )
    def sc_transpose(tab_hbm, out_hbm, in_a, in_b, out_a, out_b,
                     sem_ia, sem_ib, sem_oa, sem_ob):
        wid = lax.axis_index("s") * nc + lax.axis_index("c")
        lane = lax.iota(jnp.int32, nl)
        l_hi = lane >> 3
        l_lo = lane & 7
        t_begin = wid * tpw
        t_end = jnp.minimum(t_begin + tpw, NTC)
        ins = (in_a, in_b)
        outs = (out_a, out_b)
        sem_i = (sem_ia, sem_ib)
        sem_o = (sem_oa, sem_ob)

        def c0_of(t):
            return pl.multiple_of(t * 128, 128)

        def in_copies(b, p):
            t0 = t_begin + b * NT
            res = []
            for jt in range(NT):
                c0 = c0_of(t0 + jt)
                for dt in range(8):
                    res.append((
                        tab_hbm.at[pl.ds(dt * 8, 8), pl.ds(c0, 128)],
                        ins[p].at[jt, dt],
                    ))
            return res, t0

        def fire_in(b, p):
            pairs, t0 = in_copies(b, p)
            for jt in range(NT):
                @pl.when(t0 + jt < t_end)
                def _():
                    for src, dst in pairs[jt * 8:(jt + 1) * 8]:
                        pltpu.async_copy(src, dst, sem_i[p])

        def drain_in(b, p):
            pairs, t0 = in_copies(b, p)
            for jt in range(NT):
                @pl.when(t0 + jt < t_end)
                def _():
                    for src, dst in pairs[jt * 8:(jt + 1) * 8]:
                        pltpu.make_async_copy(src, dst, sem_i[p]).wait()

        def out_copies(b, p):
            t0 = t_begin + b * NT
            res = []
            for jt in range(NT):
                c0 = c0_of(t0 + jt)
                res.append((
                    outs[p].at[pl.ds(jt * 128 * D, 128 * D)],
                    out_hbm.at[pl.ds(c0 * D, 128 * D)],
                ))
            return res, t0

        def fire_out(b, p):
            pairs, t0 = out_copies(b, p)
            for jt in range(NT):
                @pl.when(t0 + jt < t_end)
                def _():
                    pltpu.async_copy(pairs[jt][0], pairs[jt][1], sem_o[p])

        def drain_out(b, p):
            pairs, t0 = out_copies(b, p)
            for jt in range(NT):
                @pl.when((b >= 0) & (t0 + jt < t_end))
                def _():
                    pltpu.make_async_copy(pairs[jt][0], pairs[jt][1],
                                          sem_o[p]).wait()

        def compute(p):
            lane_d = lane * D
            for jt in range(NT):
                for dt in range(8):
                    def line_body(dloc, _):
                        d = dt * 8 + dloc
                        base = lane_d + d + jt * 128 * D
                        for jl0 in range(8):
                            vals = ins[p][jt, dt, dloc,
                                          pl.ds(jl0 * nl, nl)]
                            plsc.store_scatter(
                                outs[p], [base + jl0 * nl * D], vals)
                        return 0

                    lax.fori_loop(0, 8, line_body, 0)

        fire_in(0, 0)

        def pair_body(bb, _):
            b0 = 2 * bb
            fire_in(b0 + 1, 1)
            drain_in(b0, 0)
            drain_out(b0 - 2, 0)
            compute(0)
            fire_out(b0, 0)

            @pl.when(bb < nblk // 2 - 1)
            def _():
                fire_in(b0 + 2, 0)

            drain_in(b0 + 1, 1)
            drain_out(b0 - 1, 1)
            compute(1)
            fire_out(b0 + 1, 1)
            return 0

        lax.fori_loop(0, nblk // 2, pair_body, 0)
        drain_out(nblk - 2, 0)
        drain_out(nblk - 1, 1)

    return sc_transpose


def _entity_kernel(nc, ns, nl):
    nw = nc * ns
    rows_per_w = B // nw
    cb = nl
    n_chunks = rows_per_w // cb
    g_rows = cb * N
    n_sub = g_rows // IDX_MINOR
    nq = (N + nl - 1) // nl
    tail = N - (nq - 1) * nl
    mesh = plsc.VectorSubcoreMesh(core_axis_name="c", subcore_axis_name="s")

    @functools.partial(
        pl.kernel,
        out_type=(
            jax.ShapeDtypeStruct((B * N,), jnp.float32),
            jax.ShapeDtypeStruct((B * N,), jnp.float32),
        ),
        mesh=mesh,
        compiler_params=pltpu.CompilerParams(
            needs_layout_passes=False, use_tc_tiling_on_sc=False),
        scratch_types=[
            pltpu.VMEM((2, n_sub, IDX_MINOR), jnp.int32),
            pltpu.VMEM((2, g_rows, D), jnp.float32),
            pltpu.VMEM((2, cb, D), jnp.float32),
            pltpu.VMEM((g_rows,), jnp.float32),
            pltpu.VMEM((g_rows,), jnp.float32),
            pltpu.SemaphoreType.DMA,
            pltpu.SemaphoreType.DMA,
        ],
    )
    def entity_kernel(ids_hbm, ctx_hbm, table_hbm, scores_hbm, post_hbm,
                      idx_v, rows_v, ctx_v, fs_v, fp_v, sem_a, sem_b):
        wid = lax.axis_index("s") * nc + lax.axis_index("c")
        lane = lax.iota(jnp.int32, nl)
        neg_inf = jnp.float32(-jnp.inf)
        sems = (sem_a, sem_b)

        def fire(c, p):
            pltpu.sync_copy(ids_hbm.at[pl.ds(c * n_sub, n_sub)], idx_v.at[p])
            handles = [
                pltpu.async_copy(
                    table_hbm.at[idx_v.at[p, j]],
                    rows_v.at[p, pl.ds(j * IDX_MINOR, IDX_MINOR)],
                    sems[p],
                )
                for j in range(n_sub)
            ]
            pltpu.sync_copy(ctx_hbm.at[pl.ds(c * cb, cb)], ctx_v.at[p])
            return handles

        def drain(p):
            for j in range(n_sub):
                pltpu.make_async_copy(
                    table_hbm.at[idx_v.at[p, j]],
                    rows_v.at[p, pl.ds(j * IDX_MINOR, IDX_MINOR)],
                    sems[p],
                ).wait()

        def compute_chunk(c, p):
            fbase = c * g_rows

            def row_body(b, _):
                rbase = b * N
                obase = b * N
                cvec = [ctx_v[p, b, pl.ds(k * nl, nl)]
                        for k in range(D // nl)]

                svs = []
                for q in range(nq):
                    acc = jnp.zeros((nl,), jnp.float32)
                    nlim = tail if q == nq - 1 else nl
                    for j in range(nlim):
                        r = rbase + q * nl + j
                        prod = rows_v[p, r, pl.ds(0, nl)] * cvec[0]
                        for k in range(1, D // nl):
                            prod = (prod
                                    + rows_v[p, r, pl.ds(k * nl, nl)] * cvec[k])
                        acc = jnp.where(lane == j, jnp.sum(prod), acc)
                    svs.append(acc)

                mvec = jnp.where(lane < tail, svs[nq - 1], neg_inf)
                for q in range(nq - 1):
                    mvec = jnp.maximum(mvec, svs[q])
                m = jnp.max(mvec)
                evs = [jnp.exp(sv - m) for sv in svs]
                evs[nq - 1] = jnp.where(lane < tail, evs[nq - 1], 0.0)
                ssum = evs[0]
                for q in range(1, nq):
                    ssum = ssum + evs[q]
                svec = jnp.zeros((nl,), jnp.float32) + jnp.sum(ssum)
                rinv = jnp.ones((nl,), jnp.float32) / svec

                for q in range(nq - 1):
                    fs_v[pl.ds(obase + q * nl, nl)] = svs[q]
                    fp_v[pl.ds(obase + q * nl, nl)] = evs[q] * rinv
                tmask = lane < tail
                tidx = obase + (nq - 1) * nl + lane
                plsc.store_scatter(fs_v, [tidx], svs[nq - 1], mask=tmask)
                plsc.store_scatter(fp_v, [tidx], evs[nq - 1] * rinv,
                                   mask=tmask)
                return 0

            lax.fori_loop(0, cb, row_body, 0)

            pltpu.sync_copy(fs_v, scores_hbm.at[pl.ds(fbase, g_rows)])
            pltpu.sync_copy(fp_v, post_hbm.at[pl.ds(fbase, g_rows)])

        c0_first = wid * n_chunks
        fire(c0_first, 0)

        def pair_body(gg, _):
            c = c0_first + 2 * gg
            handles_b = fire(c + 1, 1)
            drain(0)
            compute_chunk(c, 0)

            @pl.when(gg < n_chunks // 2 - 1)
            def _():
                fire(c + 2, 0)

            for h in handles_b:
                h.wait()
            compute_chunk(c + 1, 1)
            return 0

        lax.fori_loop(0, n_chunks // 2, pair_body, 0)

    return entity_kernel


def kernel(context_encoded, entity_ids, knwn_entity_embeddings):
    info = plsc.get_sparse_core_info()
    nc, ns, nl = info.num_cores, info.num_subcores, info.num_lanes
    tr = _sc_transpose(nc, ns, nl)
    table_lin = tr(knwn_entity_embeddings.T).reshape(VT, D)
    ids2d = entity_ids.astype(jnp.int32).reshape(B * N // IDX_MINOR,
                                                 IDX_MINOR)
    k = _entity_kernel(nc, ns, nl)
    scores_flat, post_flat = k(ids2d, context_encoded, table_lin)
    return scores_flat.reshape(B, N), post_flat.reshape(B, N)


# speedup vs baseline: 2.3932x; 1.9715x over previous
import functools

import jax
import jax.numpy as jnp
from jax import lax
from jax.experimental import pallas as pl
from jax.experimental.pallas import tpu as pltpu
from jax.experimental.pallas import tpu_sc as plsc

B = 4096
N = 50
D = 64
V = 1000000
TR_C = 512
TR_G = (V + 2 * TR_C - 1) // (2 * TR_C)
VP = TR_G * TR_C
IDX_MINOR = 100


def _transpose_table(table_t):
    def body(x_ref, o_ref):
        x = x_ref[...]
        o_ref[...] = jnp.concatenate(
            [x[:, :TR_C].T, x[:, TR_C:].T], axis=1)

    return pl.pallas_call(
        body,
        grid=(TR_G,),
        in_specs=[pl.BlockSpec((D, 2 * TR_C), lambda g: (0, g))],
        out_specs=pl.BlockSpec((TR_C, 2 * D), lambda g: (g, 0)),
        out_shape=jax.ShapeDtypeStruct((VP, 2 * D), jnp.float32),
    )(table_t)


def _entity_kernel(nc, ns, nl):
    nw = nc * ns
    rows_per_w = B // nw
    cb = nl
    n_chunks = rows_per_w // cb
    g_rows = cb * N
    n_sub = g_rows // IDX_MINOR
    nq = (N + nl - 1) // nl
    tail = N - (nq - 1) * nl
    mesh = plsc.VectorSubcoreMesh(core_axis_name="c", subcore_axis_name="s")

    @functools.partial(
        pl.kernel,
        out_type=(
            jax.ShapeDtypeStruct((B * N,), jnp.float32),
            jax.ShapeDtypeStruct((B * N,), jnp.float32),
        ),
        mesh=mesh,
        compiler_params=pltpu.CompilerParams(
            needs_layout_passes=False, use_tc_tiling_on_sc=False),
        scratch_types=[
            pltpu.VMEM((2, n_sub, IDX_MINOR), jnp.int32),
            pltpu.VMEM((2, g_rows, D), jnp.float32),
            pltpu.VMEM((2, cb, D), jnp.float32),
            pltpu.VMEM((g_rows,), jnp.float32),
            pltpu.VMEM((g_rows,), jnp.float32),
            pltpu.SemaphoreType.DMA,
            pltpu.SemaphoreType.DMA,
        ],
    )
    def entity_kernel(ids_hbm, ctx_hbm, table_hbm, scores_hbm, post_hbm,
                      idx_v, rows_v, ctx_v, fs_v, fp_v, sem_a, sem_b):
        wid = lax.axis_index("s") * nc + lax.axis_index("c")
        lane = lax.iota(jnp.int32, nl)
        neg_inf = jnp.float32(-jnp.inf)
        sems = (sem_a, sem_b)

        def fire(c, p):
            pltpu.sync_copy(ids_hbm.at[pl.ds(c * n_sub, n_sub)], idx_v.at[p])
            handles = [
                pltpu.async_copy(
                    table_hbm.at[idx_v.at[p, j]],
                    rows_v.at[p, pl.ds(j * IDX_MINOR, IDX_MINOR)],
                    sems[p],
                )
                for j in range(n_sub)
            ]
            pltpu.sync_copy(ctx_hbm.at[pl.ds(c * cb, cb)], ctx_v.at[p])
            return handles

        def drain(p):
            for j in range(n_sub):
                pltpu.make_async_copy(
                    table_hbm.at[idx_v.at[p, j]],
                    rows_v.at[p, pl.ds(j * IDX_MINOR, IDX_MINOR)],
                    sems[p],
                ).wait()

        def compute_chunk(c, p):
            fbase = c * g_rows

            def row_body(b, _):
                rbase = b * N
                obase = b * N
                cvec = [ctx_v[p, b, pl.ds(k * nl, nl)]
                        for k in range(D // nl)]

                svs = []
                for q in range(nq):
                    acc = jnp.zeros((nl,), jnp.float32)
                    nlim = tail if q == nq - 1 else nl
                    for j in range(nlim):
                        r = rbase + q * nl + j
                        prod = rows_v[p, r, pl.ds(0, nl)] * cvec[0]
                        for k in range(1, D // nl):
                            prod = (prod
                                    + rows_v[p, r, pl.ds(k * nl, nl)] * cvec[k])
                        acc = jnp.where(lane == j, jnp.sum(prod), acc)
                    svs.append(acc)

                mvec = jnp.where(lane < tail, svs[nq - 1], neg_inf)
                for q in range(nq - 1):
                    mvec = jnp.maximum(mvec, svs[q])
                m = jnp.max(mvec)
                evs = [jnp.exp(sv - m) for sv in svs]
                evs[nq - 1] = jnp.where(lane < tail, evs[nq - 1], 0.0)
                ssum = evs[0]
                for q in range(1, nq):
                    ssum = ssum + evs[q]
                svec = jnp.zeros((nl,), jnp.float32) + jnp.sum(ssum)
                rinv = jnp.ones((nl,), jnp.float32) / svec

                for q in range(nq - 1):
                    fs_v[pl.ds(obase + q * nl, nl)] = svs[q]
                    fp_v[pl.ds(obase + q * nl, nl)] = evs[q] * rinv
                tmask = lane < tail
                tidx = obase + (nq - 1) * nl + lane
                plsc.store_scatter(fs_v, [tidx], svs[nq - 1], mask=tmask)
                plsc.store_scatter(fp_v, [tidx], evs[nq - 1] * rinv,
                                   mask=tmask)
                return 0

            lax.fori_loop(0, cb, row_body, 0)

            pltpu.sync_copy(fs_v, scores_hbm.at[pl.ds(fbase, g_rows)])
            pltpu.sync_copy(fp_v, post_hbm.at[pl.ds(fbase, g_rows)])

        c0_first = wid * n_chunks
        fire(c0_first, 0)

        def pair_body(gg, _):
            c = c0_first + 2 * gg
            handles_b = fire(c + 1, 1)
            drain(0)
            compute_chunk(c, 0)

            @pl.when(gg < n_chunks // 2 - 1)
            def _():
                fire(c + 2, 0)

            for h in handles_b:
                h.wait()
            compute_chunk(c + 1, 1)
            return 0

        lax.fori_loop(0, n_chunks // 2, pair_body, 0)

    return entity_kernel


def kernel(context_encoded, entity_ids, knwn_entity_embeddings):
    info = plsc.get_sparse_core_info()
    nc, ns, nl = info.num_cores, info.num_subcores, info.num_lanes
    ids = entity_ids.astype(jnp.int32)
    ids2d = ids.reshape(B * N // IDX_MINOR, IDX_MINOR)
    k = _entity_kernel(nc, ns, nl)
    scores_flat, post_flat = k(ids2d, context_encoded,
                               knwn_entity_embeddings)
    return scores_flat.reshape(B, N), post_flat.reshape(B, N)


# speedup vs baseline: 2.5971x; 1.0852x over previous
import functools

import jax
import jax.numpy as jnp
from jax import lax
from jax.experimental import pallas as pl
from jax.experimental.pallas import tpu as pltpu
from jax.experimental.pallas import tpu_sc as plsc

B = 4096
N = 50
D = 64
V = 1000000
TR_C = 512
TR_G = (V + 2 * TR_C - 1) // (2 * TR_C)
VP = TR_G * TR_C
IDX_MINOR = 100


def _transpose_table(table_t):
    def body(x_ref, o_ref):
        x = x_ref[...]
        o_ref[...] = jnp.concatenate(
            [x[:, :TR_C].T, x[:, TR_C:].T], axis=1)

    return pl.pallas_call(
        body,
        grid=(TR_G,),
        in_specs=[pl.BlockSpec((D, 2 * TR_C), lambda g: (0, g))],
        out_specs=pl.BlockSpec((TR_C, 2 * D), lambda g: (g, 0)),
        out_shape=jax.ShapeDtypeStruct((VP, 2 * D), jnp.float32),
    )(table_t)


def _entity_kernel(nc, ns, nl):
    nw = nc * ns
    rows_per_w = B // nw
    cb = nl // 2
    n_chunks = rows_per_w // cb
    g_rows = cb * N
    n_sub = g_rows // IDX_MINOR
    nq = (N + nl - 1) // nl
    tail = N - (nq - 1) * nl
    mesh = plsc.VectorSubcoreMesh(core_axis_name="c", subcore_axis_name="s")

    @functools.partial(
        pl.kernel,
        out_type=(
            jax.ShapeDtypeStruct((B * N,), jnp.float32),
            jax.ShapeDtypeStruct((B * N,), jnp.float32),
        ),
        mesh=mesh,
        compiler_params=pltpu.CompilerParams(
            needs_layout_passes=False, use_tc_tiling_on_sc=False),
        scratch_types=[
            pltpu.VMEM((2, n_sub, IDX_MINOR), jnp.int32),
            pltpu.VMEM((2, g_rows, 2 * D), jnp.float32),
            pltpu.VMEM((2, cb, D), jnp.float32),
            pltpu.VMEM((g_rows,), jnp.float32),
            pltpu.VMEM((g_rows,), jnp.float32),
            pltpu.SemaphoreType.DMA,
            pltpu.SemaphoreType.DMA,
        ],
    )
    def entity_kernel(ids_hbm, ctx_hbm, table_hbm, scores_hbm, post_hbm,
                      idx_v, rows_v, ctx_v, fs_v, fp_v, sem_a, sem_b):
        wid = lax.axis_index("s") * nc + lax.axis_index("c")
        lane = lax.iota(jnp.int32, nl)
        neg_inf = jnp.float32(-jnp.inf)
        sems = (sem_a, sem_b)

        def fire(c, p):
            pltpu.sync_copy(ids_hbm.at[pl.ds(c * n_sub, n_sub)], idx_v.at[p])
            handles = [
                pltpu.async_copy(
                    table_hbm.at[idx_v.at[p, j]],
                    rows_v.at[p, pl.ds(j * IDX_MINOR, IDX_MINOR)],
                    sems[p],
                )
                for j in range(n_sub)
            ]
            pltpu.sync_copy(ctx_hbm.at[pl.ds(c * cb, cb)], ctx_v.at[p])
            return handles

        def drain(p):
            for j in range(n_sub):
                pltpu.make_async_copy(
                    table_hbm.at[idx_v.at[p, j]],
                    rows_v.at[p, pl.ds(j * IDX_MINOR, IDX_MINOR)],
                    sems[p],
                ).wait()

        def compute_chunk(c, p):
            fbase = c * g_rows

            def row_body(b, _):
                rbase = b * N
                obase = b * N
                cvec = [ctx_v[p, b, pl.ds(k * nl, nl)]
                        for k in range(D // nl)]

                svs = []
                for q in range(nq):
                    acc = jnp.zeros((nl,), jnp.float32)
                    nlim = tail if q == nq - 1 else nl
                    for j in range(nlim):
                        r = rbase + q * nl + j
                        prod = rows_v[p, r, pl.ds(0, nl)] * cvec[0]
                        for k in range(1, D // nl):
                            prod = (prod
                                    + rows_v[p, r, pl.ds(k * nl, nl)] * cvec[k])
                        acc = jnp.where(lane == j, jnp.sum(prod), acc)
                    svs.append(acc)

                mvec = jnp.where(lane < tail, svs[nq - 1], neg_inf)
                for q in range(nq - 1):
                    mvec = jnp.maximum(mvec, svs[q])
                m = jnp.max(mvec)
                evs = [jnp.exp(sv - m) for sv in svs]
                evs[nq - 1] = jnp.where(lane < tail, evs[nq - 1], 0.0)
                ssum = evs[0]
                for q in range(1, nq):
                    ssum = ssum + evs[q]
                svec = jnp.zeros((nl,), jnp.float32) + jnp.sum(ssum)
                rinv = jnp.ones((nl,), jnp.float32) / svec

                for q in range(nq - 1):
                    fs_v[pl.ds(obase + q * nl, nl)] = svs[q]
                    fp_v[pl.ds(obase + q * nl, nl)] = evs[q] * rinv
                tmask = lane < tail
                tidx = obase + (nq - 1) * nl + lane
                plsc.store_scatter(fs_v, [tidx], svs[nq - 1], mask=tmask)
                plsc.store_scatter(fp_v, [tidx], evs[nq - 1] * rinv,
                                   mask=tmask)
                return 0

            lax.fori_loop(0, cb, row_body, 0)

            pltpu.sync_copy(fs_v, scores_hbm.at[pl.ds(fbase, g_rows)])
            pltpu.sync_copy(fp_v, post_hbm.at[pl.ds(fbase, g_rows)])

        c0_first = wid * n_chunks
        fire(c0_first, 0)

        def pair_body(gg, _):
            c = c0_first + 2 * gg
            handles_b = fire(c + 1, 1)
            drain(0)
            compute_chunk(c, 0)

            @pl.when(gg < n_chunks // 2 - 1)
            def _():
                fire(c + 2, 0)

            for h in handles_b:
                h.wait()
            compute_chunk(c + 1, 1)
            return 0

        lax.fori_loop(0, n_chunks // 2, pair_body, 0)

    return entity_kernel


def kernel(context_encoded, entity_ids, knwn_entity_embeddings):
    info = plsc.get_sparse_core_info()
    nc, ns, nl = info.num_cores, info.num_subcores, info.num_lanes
    ids = entity_ids.astype(jnp.int32)
    ids2d = ids.reshape(B * N // IDX_MINOR, IDX_MINOR)
    table128 = jnp.pad(knwn_entity_embeddings, ((0, 0), (0, D)))
    k = _entity_kernel(nc, ns, nl)
    scores_flat, post_flat = k(ids2d, context_encoded, table128)
    return scores_flat.reshape(B, N), post_flat.reshape(B, N)


# speedup vs baseline: 2.6026x; 1.0021x over previous
import functools

import jax
import jax.numpy as jnp
from jax import lax
from jax.experimental import pallas as pl
from jax.experimental.pallas import tpu as pltpu
from jax.experimental.pallas import tpu_sc as plsc

B = 4096
N = 50
D = 64
V = 1000000
IDX_MINOR = 100


def _entity_kernel(nc, ns, nl):
    nw = nc * ns
    rows_per_w = B // nw
    cb = nl // 2
    n_chunks = rows_per_w // cb
    g_rows = cb * N
    n_sub = g_rows // IDX_MINOR
    nq = (N + nl - 1) // nl
    tail = N - (nq - 1) * nl
    mesh = plsc.VectorSubcoreMesh(core_axis_name="c", subcore_axis_name="s")

    @functools.partial(
        pl.kernel,
        out_type=(
            jax.ShapeDtypeStruct((B * N,), jnp.float32),
            jax.ShapeDtypeStruct((B * N,), jnp.float32),
        ),
        mesh=mesh,
        compiler_params=pltpu.CompilerParams(
            needs_layout_passes=False, use_tc_tiling_on_sc=False),
        scratch_types=[
            pltpu.VMEM((2, n_sub, IDX_MINOR), jnp.int32),
            pltpu.VMEM((2, g_rows, 2 * D), jnp.float32),
            pltpu.VMEM((2, cb, D), jnp.float32),
            pltpu.VMEM((g_rows,), jnp.float32),
            pltpu.VMEM((g_rows,), jnp.float32),
            pltpu.SemaphoreType.DMA,
            pltpu.SemaphoreType.DMA,
        ],
    )
    def entity_kernel(ids_hbm, ctx_hbm, table_hbm, scores_hbm, post_hbm,
                      idx_v, rows_v, ctx_v, fs_v, fp_v, sem_a, sem_b):
        wid = lax.axis_index("s") * nc + lax.axis_index("c")
        lane = lax.iota(jnp.int32, nl)
        neg_inf = jnp.float32(-jnp.inf)
        sems = (sem_a, sem_b)

        def fire(c, p):
            pltpu.sync_copy(ids_hbm.at[pl.ds(c * n_sub, n_sub)], idx_v.at[p])
            handles = [
                pltpu.async_copy(
                    table_hbm.at[idx_v.at[p, j]],
                    rows_v.at[p, pl.ds(j * IDX_MINOR, IDX_MINOR)],
                    sems[p],
                )
                for j in range(n_sub)
            ]
            pltpu.sync_copy(ctx_hbm.at[pl.ds(c * cb, cb)], ctx_v.at[p])
            return handles

        def drain(p):
            for j in range(n_sub):
                pltpu.make_async_copy(
                    table_hbm.at[idx_v.at[p, j]],
                    rows_v.at[p, pl.ds(j * IDX_MINOR, IDX_MINOR)],
                    sems[p],
                ).wait()

        def compute_chunk(c, p):
            fbase = c * g_rows

            def row_body(b, _):
                rbase = b * N
                obase = b * N
                cvec = [ctx_v[p, b, pl.ds(k * nl, nl)]
                        for k in range(D // nl)]

                svs = []
                for q in range(nq):
                    acc = jnp.zeros((nl,), jnp.float32)
                    nlim = tail if q == nq - 1 else nl
                    for j in range(nlim):
                        r = rbase + q * nl + j
                        prod = rows_v[p, r, pl.ds(0, nl)] * cvec[0]
                        for k in range(1, D // nl):
                            prod = (prod
                                    + rows_v[p, r, pl.ds(k * nl, nl)] * cvec[k])
                        acc = jnp.where(lane == j, jnp.sum(prod), acc)
                    svs.append(acc)

                mvec = jnp.where(lane < tail, svs[nq - 1], neg_inf)
                for q in range(nq - 1):
                    mvec = jnp.maximum(mvec, svs[q])
                m = jnp.max(mvec)
                evs = [jnp.exp(sv - m) for sv in svs]
                evs[nq - 1] = jnp.where(lane < tail, evs[nq - 1], 0.0)
                ssum = evs[0]
                for q in range(1, nq):
                    ssum = ssum + evs[q]
                svec = jnp.zeros((nl,), jnp.float32) + jnp.sum(ssum)
                rinv = jnp.ones((nl,), jnp.float32) / svec

                for q in range(nq - 1):
                    fs_v[pl.ds(obase + q * nl, nl)] = svs[q]
                    fp_v[pl.ds(obase + q * nl, nl)] = evs[q] * rinv
                tmask = lane < tail
                tidx = obase + (nq - 1) * nl + lane
                plsc.store_scatter(fs_v, [tidx], svs[nq - 1], mask=tmask)
                plsc.store_scatter(fp_v, [tidx], evs[nq - 1] * rinv,
                                   mask=tmask)
                return 0

            lax.fori_loop(0, cb, row_body, 0)

            pltpu.sync_copy(fs_v, scores_hbm.at[pl.ds(fbase, g_rows)])
            pltpu.sync_copy(fp_v, post_hbm.at[pl.ds(fbase, g_rows)])

        c0_first = wid * n_chunks
        fire(c0_first, 0)

        def pair_body(gg, _):
            c = c0_first + 2 * gg
            handles_b = fire(c + 1, 1)
            drain(0)
            compute_chunk(c, 0)

            @pl.when(gg < n_chunks // 2 - 1)
            def _():
                fire(c + 2, 0)

            for h in handles_b:
                h.wait()
            compute_chunk(c + 1, 1)
            return 0

        lax.fori_loop(0, n_chunks // 2, pair_body, 0)

    return entity_kernel


def kernel(context_encoded, entity_ids, knwn_entity_embeddings):
    info = plsc.get_sparse_core_info()
    nc, ns, nl = info.num_cores, info.num_subcores, info.num_lanes
    ids = entity_ids.astype(jnp.int32)
    ids2d = ids.reshape(B * N // IDX_MINOR, IDX_MINOR)
    table128 = jnp.pad(knwn_entity_embeddings, ((0, 0), (0, D)))
    k = _entity_kernel(nc, ns, nl)
    scores_flat, post_flat = k(ids2d, context_encoded, table128)
    return scores_flat.reshape(B, N), post_flat.reshape(B, N)
